# baseline (speedup 1.0000x reference)
import functools

import jax
import jax.numpy as jnp
from jax import lax
from jax.experimental import pallas as pl
from jax.experimental.pallas import tpu as pltpu
from jax.experimental.pallas import tpu_sc as plsc

N = 10000
E = 160000
D_IN = 384
D_HID = 128

_BLK = 1000
_GRID = N // _BLK
_K = 80
_EPT = E // 16
_CH = _EPT // _K
_WCH = 400
_NWC = N // _WCH


def _leaky(z):
    return jnp.where(z >= 0, z, 0.2 * z)


def _tc_pre_body(x_ref, keep_ref, mtok_ref, w_ref, as_ref, ad_ref,
                 hc_ref, a1_ref, a2_ref):
    keep = keep_ref[...]
    xm = keep * x_ref[...] + (1.0 - keep) * mtok_ref[...]
    h = jnp.dot(xm, w_ref[...], preferred_element_type=jnp.float32)
    hc_ref[0] = h[:, : D_HID // 2]
    hc_ref[1] = h[:, D_HID // 2:]
    a1_ref[...] = jnp.sum(h * as_ref[...], axis=1, keepdims=True)
    a2_ref[...] = jnp.sum(h * ad_ref[...], axis=1, keepdims=True)


def _tc_pre(x, keepf, mtok, enc_W, a_s, a_d):
    return pl.pallas_call(
        _tc_pre_body,
        grid=(_GRID,),
        in_specs=[
            pl.BlockSpec((_BLK, D_IN), lambda i: (i, 0)),
            pl.BlockSpec((_BLK, 1), lambda i: (i, 0)),
            pl.BlockSpec((1, D_IN), lambda i: (0, 0)),
            pl.BlockSpec((D_IN, D_HID), lambda i: (0, 0)),
            pl.BlockSpec((1, D_HID), lambda i: (0, 0)),
            pl.BlockSpec((1, D_HID), lambda i: (0, 0)),
        ],
        out_specs=[
            pl.BlockSpec((2, _BLK, D_HID // 2), lambda i: (0, i, 0)),
            pl.BlockSpec((_BLK, 1), lambda i: (i, 0)),
            pl.BlockSpec((_BLK, 1), lambda i: (i, 0)),
        ],
        out_shape=[
            jax.ShapeDtypeStruct((2, N, D_HID // 2), jnp.float32),
            jax.ShapeDtypeStruct((N, 1), jnp.float32),
            jax.ShapeDtypeStruct((N, 1), jnp.float32),
        ],
    )(x, keepf, mtok, enc_W, a_s, a_d)


def _tc_mid_body(acc_ref, den_ref, hc_ref, a1_ref, a2_ref, keep_ref,
                 bias_ref, e2dt_ref, decw_ref, das_ref, dad_ref,
                 h3_ref, b1_ref, b2_ref):
    wself = jnp.exp(_leaky(a1_ref[...] + a2_ref[...]))
    h1 = jnp.concatenate([hc_ref[0], hc_ref[1]], axis=1)
    acc = jnp.concatenate([acc_ref[0], acc_ref[1]], axis=1) + wself * h1
    he = acc / (den_ref[...] + wself + 1e-16) + bias_ref[...]
    hm = keep_ref[...] * he
    h2 = jnp.dot(hm, e2dt_ref[...], preferred_element_type=jnp.float32)
    h3 = jnp.dot(h2, decw_ref[...], preferred_element_type=jnp.float32)
    for q in range(4):
        h3_ref[q] = h3[:, q * (D_IN // 4): (q + 1) * (D_IN // 4)]
    b1_ref[...] = jnp.sum(h3 * das_ref[...], axis=1, keepdims=True)
    b2_ref[...] = jnp.sum(h3 * dad_ref[...], axis=1, keepdims=True)


def _tc_mid(acc1, den1, hc1, a1, a2, keepf, enc_bias, e2d_Wt, dec_W, d_as, d_ad):
    return pl.pallas_call(
        _tc_mid_body,
        grid=(_GRID,),
        in_specs=[
            pl.BlockSpec((2, _BLK, D_HID // 2), lambda i: (0, i, 0)),
            pl.BlockSpec((_BLK, 1), lambda i: (i, 0)),
            pl.BlockSpec((2, _BLK, D_HID // 2), lambda i: (0, i, 0)),
            pl.BlockSpec((_BLK, 1), lambda i: (i, 0)),
            pl.BlockSpec((_BLK, 1), lambda i: (i, 0)),
            pl.BlockSpec((_BLK, 1), lambda i: (i, 0)),
            pl.BlockSpec((1, D_HID), lambda i: (0, 0)),
            pl.BlockSpec((D_HID, D_HID), lambda i: (0, 0)),
            pl.BlockSpec((D_HID, D_IN), lambda i: (0, 0)),
            pl.BlockSpec((1, D_IN), lambda i: (0, 0)),
            pl.BlockSpec((1, D_IN), lambda i: (0, 0)),
        ],
        out_specs=[
            pl.BlockSpec((4, _BLK, D_IN // 4), lambda i: (0, i, 0)),
            pl.BlockSpec((_BLK, 1), lambda i: (i, 0)),
            pl.BlockSpec((_BLK, 1), lambda i: (i, 0)),
        ],
        out_shape=[
            jax.ShapeDtypeStruct((4, N, D_IN // 4), jnp.float32),
            jax.ShapeDtypeStruct((N, 1), jnp.float32),
            jax.ShapeDtypeStruct((N, 1), jnp.float32),
        ],
    )(acc1, den1, hc1, a1, a2, keepf, enc_bias, e2d_Wt, dec_W, d_as, d_ad)


def _tc_loss_body(acca_ref, accb_ref, den_ref, h3_ref, b1_ref, b2_ref,
                  keep_ref, bias_ref, x_ref, loss_ref, s_ref):
    i = pl.program_id(0)

    @pl.when(i == 0)
    def _():
        s_ref[0] = 0.0
        s_ref[1] = 0.0

    wself = jnp.exp(_leaky(b1_ref[...] + b2_ref[...]))
    h3 = jnp.concatenate([h3_ref[q] for q in range(4)], axis=1)
    acc = jnp.concatenate(
        [acca_ref[0], acca_ref[1], accb_ref[0], accb_ref[1]], axis=1
    ) + wself * h3
    xr = acc / (den_ref[...] + wself + 1e-16) + bias_ref[...]
    x = x_ref[...]
    nx = jnp.maximum(jnp.sqrt(jnp.sum(xr * xr, axis=1, keepdims=True)), 1e-12)
    ny = jnp.maximum(jnp.sqrt(jnp.sum(x * x, axis=1, keepdims=True)), 1e-12)
    dot = jnp.sum(xr * x, axis=1, keepdims=True) / (nx * ny)
    z = 1.0 - dot
    mf = 1.0 - keep_ref[...]
    per = z * z * z * mf
    s_ref[0] += jnp.sum(per)
    s_ref[1] += jnp.sum(mf)

    @pl.when(i == _GRID - 1)
    def _():
        loss_ref[...] = jnp.reshape(
            s_ref[0] / jnp.maximum(s_ref[1], 1.0), (1, 1))


def _tc_loss(acc2a, acc2b, den2, hc3, b1, b2, keepf, dec_bias, x):
    return pl.pallas_call(
        _tc_loss_body,
        grid=(_GRID,),
        in_specs=[
            pl.BlockSpec((2, _BLK, D_IN // 4), lambda i: (0, i, 0)),
            pl.BlockSpec((2, _BLK, D_IN // 4), lambda i: (0, i, 0)),
            pl.BlockSpec((_BLK, 1), lambda i: (i, 0)),
            pl.BlockSpec((4, _BLK, D_IN // 4), lambda i: (0, i, 0)),
            pl.BlockSpec((_BLK, 1), lambda i: (i, 0)),
            pl.BlockSpec((_BLK, 1), lambda i: (i, 0)),
            pl.BlockSpec((_BLK, 1), lambda i: (i, 0)),
            pl.BlockSpec((1, D_IN), lambda i: (0, 0)),
            pl.BlockSpec((_BLK, D_IN), lambda i: (i, 0)),
        ],
        out_specs=pl.BlockSpec((1, 1), lambda i: (0, 0)),
        out_shape=jax.ShapeDtypeStruct((1, 1), jnp.float32),
        scratch_shapes=[pltpu.SMEM((2,), jnp.float32)],
    )(acc2a, acc2b, den2, hc3, b1, b2, keepf, dec_bias, x)


def _make_sc_edge(d_half):
    nv = d_half // 16
    nb = 3
    mesh = plsc.VectorSubcoreMesh(core_axis_name="c", subcore_axis_name="s")

    scratch = [
        pltpu.VMEM((_EPT,), jnp.int32),
        pltpu.VMEM((_EPT,), jnp.int32),
        pltpu.VMEM_SHARED((N, d_half), jnp.float32),
        pltpu.VMEM_SHARED((N,), jnp.float32),
    ]
    for _ in range(nb):
        scratch += [
            pltpu.VMEM((_K,), jnp.int32),
            pltpu.VMEM((_K,), jnp.int32),
            pltpu.VMEM((_K,), jnp.int32),
            pltpu.VMEM((_K,), jnp.float32),
            pltpu.VMEM((_K,), jnp.float32),
            pltpu.VMEM((_K,), jnp.float32),
            pltpu.VMEM((_K, d_half), jnp.float32),
            pltpu.VMEM((_K, d_half), jnp.float32),
            pltpu.SemaphoreType.DMA,
            pltpu.SemaphoreType.DMA,
            pltpu.SemaphoreType.DMA,
        ]

    @functools.partial(
        pl.kernel,
        mesh=mesh,
        compiler_params=pltpu.CompilerParams(
            needs_layout_passes=False, use_tc_tiling_on_sc=False),
        out_type=[
            jax.ShapeDtypeStruct((2 * N, d_half), jnp.float32),
            jax.ShapeDtypeStruct((2 * N,), jnp.float32),
        ],
        scratch_types=scratch,
    )
    def sc_edge(src_h, dst_h, as_h, ad_h, hcomb_h, acc_h, den_h,
                src_t, dst_t, acc_sh, den_sh, *bufs):
        c = lax.axis_index("c")
        s = lax.axis_index("s")
        sg_b = [bufs[11 * b + 0] for b in range(nb)]
        srcu_b = [bufs[11 * b + 1] for b in range(nb)]
        dst_b = [bufs[11 * b + 2] for b in range(nb)]
        asv_b = [bufs[11 * b + 3] for b in range(nb)]
        adv_b = [bufs[11 * b + 4] for b in range(nb)]
        w_b = [bufs[11 * b + 5] for b in range(nb)]
        rows_b = [bufs[11 * b + 6] for b in range(nb)]
        out_b = [bufs[11 * b + 7] for b in range(nb)]
        gsem = [bufs[11 * b + 8] for b in range(nb)]
        lsem = [bufs[11 * b + 9] for b in range(nb)]
        ssem = [bufs[11 * b + 10] for b in range(nb)]

        ebase = s * _EPT
        pltpu.sync_copy(src_h.at[pl.ds(ebase, _EPT)], src_t)
        pltpu.sync_copy(dst_h.at[pl.ds(ebase, _EPT)], dst_t)

        zf = jnp.zeros((16,), jnp.float32)
        for r in range(_K):
            for v in range(nv):
                rows_b[0][r, pl.ds(v * 16, 16)] = zf
        for g in range(_K // 16):
            w_b[0][pl.ds(g * 16, 16)] = zf
        for cc in range(_NWC):
            @pl.when(s == cc % 16)
            def _():
                for j in range(_WCH // _K):
                    pltpu.sync_copy(
                        rows_b[0],
                        acc_sh.at[pl.ds(cc * _WCH + j * _K, _K)])
                    pltpu.sync_copy(
                        w_b[0],
                        den_sh.at[pl.ds(cc * _WCH + j * _K, _K)])
        plsc.subcore_barrier()

        off = jnp.full((16,), 0, jnp.int32) + c * N

        def start_chunk(ci, b):
            for g in range(_K // 16):
                sl = pl.ds(ci * _K + g * 16, 16)
                s16 = src_t[sl]
                sg_b[b][pl.ds(g * 16, 16)] = s16 + off
                srcu_b[b][pl.ds(g * 16, 16)] = s16
                dst_b[b][pl.ds(g * 16, 16)] = dst_t[sl]
            pltpu.async_copy(hcomb_h.at[sg_b[b]], rows_b[b], gsem[b])
            pltpu.async_copy(as_h.at[srcu_b[b]], asv_b[b], lsem[b])
            pltpu.async_copy(ad_h.at[dst_b[b]], adv_b[b], lsem[b])

        def wait_scatter(b):
            pltpu.make_async_copy(
                out_b[b], acc_sh.at[dst_b[b]], ssem[b]).wait()

            @pl.when(c == 0)
            def _():
                pltpu.make_async_copy(
                    w_b[b], den_sh.at[dst_b[b]], ssem[b]).wait()

        start_chunk(0, 0)
        start_chunk(1, 1)

        def it_body(it, carry):
            for b in range(nb):
                i = it * nb + b

                @pl.when(i < _CH)
                def _():
                    pltpu.make_async_copy(
                        as_h.at[srcu_b[b]], asv_b[b], lsem[b]).wait()
                    pltpu.make_async_copy(
                        ad_h.at[dst_b[b]], adv_b[b], lsem[b]).wait()
                    for g in range(_K // 16):
                        sl = pl.ds(g * 16, 16)
                        z = asv_b[b][sl] + adv_b[b][sl]
                        w_b[b][sl] = jnp.exp(_leaky(z))
                    pltpu.make_async_copy(
                        hcomb_h.at[sg_b[b]], rows_b[b], gsem[b]).wait()
                    for e_i in range(_K):
                        wspl = plsc.load_gather(
                            w_b[b], [jnp.full((16,), e_i, jnp.int32)])
                        for v in range(nv):
                            out_b[b][e_i, pl.ds(v * 16, 16)] = (
                                rows_b[b][e_i, pl.ds(v * 16, 16)] * wspl)
                    b2 = (b + 2) % nb

                    @pl.when((i >= 1) & (i + 2 < _CH))
                    def _():
                        wait_scatter(b2)
                        start_chunk(i + 2, b2)

                    @pl.when(i == 0)
                    def _():
                        start_chunk(2, b2)

                    pltpu.async_copy(
                        out_b[b], acc_sh.at[dst_b[b]], ssem[b], add=True)

                    @pl.when(c == 0)
                    def _():
                        pltpu.async_copy(
                            w_b[b], den_sh.at[dst_b[b]], ssem[b], add=True)
            return carry

        lax.fori_loop(0, (_CH + nb - 1) // nb, it_body, 0)
        for b in range(nb):
            wait_scatter(b)
        plsc.subcore_barrier()

        for cc in range(_NWC):
            @pl.when(s == cc % 16)
            def _():
                pltpu.sync_copy(
                    acc_sh.at[pl.ds(cc * _WCH, _WCH)],
                    acc_h.at[pl.ds(c * N + cc * _WCH, _WCH)])
                pltpu.sync_copy(
                    den_sh.at[pl.ds(cc * _WCH, _WCH)],
                    den_h.at[pl.ds(c * N + cc * _WCH, _WCH)])

    return sc_edge


_sc_edge_enc = _make_sc_edge(D_HID // 2)
_sc_edge_dec = _make_sc_edge(D_IN // 4)


def kernel(x, edges, node_mask, enc_W, enc_a_src, enc_a_dst, enc_bias,
           e2d_W, mask_token, dec_W, dec_a_src, dec_a_dst, dec_bias):
    src = edges[:, 0].astype(jnp.int32)
    dst = edges[:, 1].astype(jnp.int32)
    keepf = node_mask.astype(jnp.float32)[:, None]

    hc1, a1, a2 = _tc_pre(x, keepf, mask_token, enc_W,
                          enc_a_src.reshape(1, D_HID),
                          enc_a_dst.reshape(1, D_HID))
    acc1, den1 = _sc_edge_enc(src, dst, a1.reshape(N), a2.reshape(N),
                              hc1.reshape(2 * N, D_HID // 2))
    hc3, b1, b2 = _tc_mid(acc1.reshape(2, N, D_HID // 2),
                          den1[:N].reshape(N, 1), hc1, a1, a2, keepf,
                          enc_bias.reshape(1, D_HID), e2d_W.T, dec_W,
                          dec_a_src.reshape(1, D_IN),
                          dec_a_dst.reshape(1, D_IN))
    b1f = b1.reshape(N)
    b2f = b2.reshape(N)
    acc2a, den2 = _sc_edge_dec(src, dst, b1f, b2f,
                               hc3[:2].reshape(2 * N, D_IN // 4))
    acc2b, _ = _sc_edge_dec(src, dst, b1f, b2f,
                            hc3[2:].reshape(2 * N, D_IN // 4))
    loss = _tc_loss(acc2a.reshape(2, N, D_IN // 4),
                    acc2b.reshape(2, N, D_IN // 4),
                    den2[:N].reshape(N, 1),
                    hc3, b1, b2, keepf, dec_bias.reshape(1, D_IN), x)
    return loss.reshape(())


# speedup vs baseline: 1.6571x; 1.6571x over previous
import functools

import jax
import jax.numpy as jnp
from jax import lax
from jax.experimental import pallas as pl
from jax.experimental.pallas import tpu as pltpu
from jax.experimental.pallas import tpu_sc as plsc

N = 10000
E = 160000
D_IN = 384
D_HID = 128

_BLK = 1000
_GRID = N // _BLK
_K = 80
_EPT = E // 16
_CH = _EPT // _K
_WCH = 400
_NWC = N // _WCH


def _leaky(z):
    return jnp.where(z >= 0, z, 0.2 * z)


def _tc_pre_body(x_ref, keep_ref, mtok_ref, w_ref, as_ref, ad_ref,
                 hc_ref, a1_ref, a2_ref):
    keep = keep_ref[...]
    xm = keep * x_ref[...] + (1.0 - keep) * mtok_ref[...]
    h = jnp.dot(xm, w_ref[...], preferred_element_type=jnp.float32)
    hc_ref[0] = h[:, : D_HID // 2]
    hc_ref[1] = h[:, D_HID // 2:]
    a1_ref[...] = jnp.sum(h * as_ref[...], axis=1, keepdims=True)
    a2_ref[...] = jnp.sum(h * ad_ref[...], axis=1, keepdims=True)


def _tc_pre(x, keepf, mtok, enc_W, a_s, a_d):
    return pl.pallas_call(
        _tc_pre_body,
        grid=(_GRID,),
        in_specs=[
            pl.BlockSpec((_BLK, D_IN), lambda i: (i, 0)),
            pl.BlockSpec((_BLK, 1), lambda i: (i, 0)),
            pl.BlockSpec((1, D_IN), lambda i: (0, 0)),
            pl.BlockSpec((D_IN, D_HID), lambda i: (0, 0)),
            pl.BlockSpec((1, D_HID), lambda i: (0, 0)),
            pl.BlockSpec((1, D_HID), lambda i: (0, 0)),
        ],
        out_specs=[
            pl.BlockSpec((2, _BLK, D_HID // 2), lambda i: (0, i, 0)),
            pl.BlockSpec((_BLK, 1), lambda i: (i, 0)),
            pl.BlockSpec((_BLK, 1), lambda i: (i, 0)),
        ],
        out_shape=[
            jax.ShapeDtypeStruct((2, N, D_HID // 2), jnp.float32),
            jax.ShapeDtypeStruct((N, 1), jnp.float32),
            jax.ShapeDtypeStruct((N, 1), jnp.float32),
        ],
    )(x, keepf, mtok, enc_W, a_s, a_d)


def _tc_mid_body(acc_ref, den_ref, hc_ref, a1_ref, a2_ref, keep_ref,
                 bias_ref, e2dt_ref, decw_ref, das_ref, dad_ref,
                 h3_ref, b1_ref, b2_ref):
    wself = jnp.exp(_leaky(a1_ref[...] + a2_ref[...]))
    h1 = jnp.concatenate([hc_ref[0], hc_ref[1]], axis=1)
    acc = jnp.concatenate([acc_ref[0], acc_ref[1]], axis=1) + wself * h1
    he = acc / (den_ref[...] + wself + 1e-16) + bias_ref[...]
    hm = keep_ref[...] * he
    h2 = jnp.dot(hm, e2dt_ref[...], preferred_element_type=jnp.float32)
    h3 = jnp.dot(h2, decw_ref[...], preferred_element_type=jnp.float32)
    for q in range(4):
        h3_ref[q] = h3[:, q * (D_IN // 4): (q + 1) * (D_IN // 4)]
    b1_ref[...] = jnp.sum(h3 * das_ref[...], axis=1, keepdims=True)
    b2_ref[...] = jnp.sum(h3 * dad_ref[...], axis=1, keepdims=True)


def _tc_mid(acc1, den1, hc1, a1, a2, keepf, enc_bias, e2d_Wt, dec_W, d_as, d_ad):
    return pl.pallas_call(
        _tc_mid_body,
        grid=(_GRID,),
        in_specs=[
            pl.BlockSpec((2, _BLK, D_HID // 2), lambda i: (0, i, 0)),
            pl.BlockSpec((_BLK, 1), lambda i: (i, 0)),
            pl.BlockSpec((2, _BLK, D_HID // 2), lambda i: (0, i, 0)),
            pl.BlockSpec((_BLK, 1), lambda i: (i, 0)),
            pl.BlockSpec((_BLK, 1), lambda i: (i, 0)),
            pl.BlockSpec((_BLK, 1), lambda i: (i, 0)),
            pl.BlockSpec((1, D_HID), lambda i: (0, 0)),
            pl.BlockSpec((D_HID, D_HID), lambda i: (0, 0)),
            pl.BlockSpec((D_HID, D_IN), lambda i: (0, 0)),
            pl.BlockSpec((1, D_IN), lambda i: (0, 0)),
            pl.BlockSpec((1, D_IN), lambda i: (0, 0)),
        ],
        out_specs=[
            pl.BlockSpec((4, _BLK, D_IN // 4), lambda i: (0, i, 0)),
            pl.BlockSpec((_BLK, 1), lambda i: (i, 0)),
            pl.BlockSpec((_BLK, 1), lambda i: (i, 0)),
        ],
        out_shape=[
            jax.ShapeDtypeStruct((4, N, D_IN // 4), jnp.float32),
            jax.ShapeDtypeStruct((N, 1), jnp.float32),
            jax.ShapeDtypeStruct((N, 1), jnp.float32),
        ],
    )(acc1, den1, hc1, a1, a2, keepf, enc_bias, e2d_Wt, dec_W, d_as, d_ad)


def _tc_loss_body(acca_ref, accb_ref, den_ref, h3_ref, b1_ref, b2_ref,
                  keep_ref, bias_ref, x_ref, loss_ref, s_ref):
    i = pl.program_id(0)

    @pl.when(i == 0)
    def _():
        s_ref[0] = 0.0
        s_ref[1] = 0.0

    wself = jnp.exp(_leaky(b1_ref[...] + b2_ref[...]))
    h3 = jnp.concatenate([h3_ref[q] for q in range(4)], axis=1)
    acc = jnp.concatenate(
        [acca_ref[0], acca_ref[1], accb_ref[0], accb_ref[1]], axis=1
    ) + wself * h3
    xr = acc / (den_ref[...] + wself + 1e-16) + bias_ref[...]
    x = x_ref[...]
    nx = jnp.maximum(jnp.sqrt(jnp.sum(xr * xr, axis=1, keepdims=True)), 1e-12)
    ny = jnp.maximum(jnp.sqrt(jnp.sum(x * x, axis=1, keepdims=True)), 1e-12)
    dot = jnp.sum(xr * x, axis=1, keepdims=True) / (nx * ny)
    z = 1.0 - dot
    mf = 1.0 - keep_ref[...]
    per = z * z * z * mf
    s_ref[0] += jnp.sum(per)
    s_ref[1] += jnp.sum(mf)

    @pl.when(i == _GRID - 1)
    def _():
        loss_ref[...] = jnp.reshape(
            s_ref[0] / jnp.maximum(s_ref[1], 1.0), (1, 1))


def _tc_loss(acc2a, acc2b, den2, hc3, b1, b2, keepf, dec_bias, x):
    return pl.pallas_call(
        _tc_loss_body,
        grid=(_GRID,),
        in_specs=[
            pl.BlockSpec((2, _BLK, D_IN // 4), lambda i: (0, i, 0)),
            pl.BlockSpec((2, _BLK, D_IN // 4), lambda i: (0, i, 0)),
            pl.BlockSpec((_BLK, 1), lambda i: (i, 0)),
            pl.BlockSpec((4, _BLK, D_IN // 4), lambda i: (0, i, 0)),
            pl.BlockSpec((_BLK, 1), lambda i: (i, 0)),
            pl.BlockSpec((_BLK, 1), lambda i: (i, 0)),
            pl.BlockSpec((_BLK, 1), lambda i: (i, 0)),
            pl.BlockSpec((1, D_IN), lambda i: (0, 0)),
            pl.BlockSpec((_BLK, D_IN), lambda i: (i, 0)),
        ],
        out_specs=pl.BlockSpec((1, 1), lambda i: (0, 0)),
        out_shape=jax.ShapeDtypeStruct((1, 1), jnp.float32),
        scratch_shapes=[pltpu.SMEM((2,), jnp.float32)],
    )(acc2a, acc2b, den2, hc3, b1, b2, keepf, dec_bias, x)


def _make_sc_edge(d_half):
    nv = d_half // 16
    nb = 3
    mesh = plsc.VectorSubcoreMesh(core_axis_name="c", subcore_axis_name="s")

    scratch = [
        pltpu.VMEM((_EPT,), jnp.int32),
        pltpu.VMEM((_EPT,), jnp.int32),
        pltpu.VMEM_SHARED((N, d_half), jnp.float32),
        pltpu.VMEM_SHARED((N,), jnp.float32),
    ]
    for _ in range(nb):
        scratch += [
            pltpu.VMEM((_K,), jnp.int32),
            pltpu.VMEM((_K,), jnp.int32),
            pltpu.VMEM((_K,), jnp.int32),
            pltpu.VMEM((_K,), jnp.float32),
            pltpu.VMEM((_K,), jnp.float32),
            pltpu.VMEM((_K,), jnp.float32),
            pltpu.VMEM((_K, d_half), jnp.float32),
            pltpu.VMEM((_K, d_half), jnp.float32),
            pltpu.SemaphoreType.DMA,
            pltpu.SemaphoreType.DMA,
            pltpu.SemaphoreType.DMA,
        ]

    @functools.partial(
        pl.kernel,
        mesh=mesh,
        compiler_params=pltpu.CompilerParams(
            needs_layout_passes=False, use_tc_tiling_on_sc=False),
        out_type=[
            jax.ShapeDtypeStruct((2 * N, d_half), jnp.float32),
            jax.ShapeDtypeStruct((2 * N,), jnp.float32),
        ],
        scratch_types=scratch,
    )
    def sc_edge(src_h, dst_h, as_h, ad_h, hcomb_h, acc_h, den_h,
                src_t, dst_t, acc_sh, den_sh, *bufs):
        c = lax.axis_index("c")
        s = lax.axis_index("s")
        sg_b = [bufs[11 * b + 0] for b in range(nb)]
        srcu_b = [bufs[11 * b + 1] for b in range(nb)]
        dst_b = [bufs[11 * b + 2] for b in range(nb)]
        asv_b = [bufs[11 * b + 3] for b in range(nb)]
        adv_b = [bufs[11 * b + 4] for b in range(nb)]
        w_b = [bufs[11 * b + 5] for b in range(nb)]
        rows_b = [bufs[11 * b + 6] for b in range(nb)]
        out_b = [bufs[11 * b + 7] for b in range(nb)]
        gsem = [bufs[11 * b + 8] for b in range(nb)]
        lsem = [bufs[11 * b + 9] for b in range(nb)]
        ssem = [bufs[11 * b + 10] for b in range(nb)]

        ebase = s * _EPT
        pltpu.sync_copy(src_h.at[pl.ds(ebase, _EPT)], src_t)
        pltpu.sync_copy(dst_h.at[pl.ds(ebase, _EPT)], dst_t)

        zf = jnp.zeros((16,), jnp.float32)
        for r in range(_K):
            for v in range(nv):
                rows_b[0][r, pl.ds(v * 16, 16)] = zf
        for g in range(_K // 16):
            w_b[0][pl.ds(g * 16, 16)] = zf
        for cc in range(_NWC):
            @pl.when(s == cc % 16)
            def _():
                for j in range(_WCH // _K):
                    pltpu.sync_copy(
                        rows_b[0],
                        acc_sh.at[pl.ds(cc * _WCH + j * _K, _K)])
                    pltpu.sync_copy(
                        w_b[0],
                        den_sh.at[pl.ds(cc * _WCH + j * _K, _K)])
        plsc.subcore_barrier()

        off = jnp.full((16,), 0, jnp.int32) + c * N

        def start_chunk(ci, b):
            for g in range(_K // 16):
                sl = pl.ds(ci * _K + g * 16, 16)
                s16 = src_t[sl]
                sg_b[b][pl.ds(g * 16, 16)] = s16 + off
                srcu_b[b][pl.ds(g * 16, 16)] = s16
                dst_b[b][pl.ds(g * 16, 16)] = dst_t[sl]
            pltpu.async_copy(hcomb_h.at[sg_b[b]], rows_b[b], gsem[b])
            pltpu.async_copy(as_h.at[srcu_b[b]], asv_b[b], lsem[b])
            pltpu.async_copy(ad_h.at[dst_b[b]], adv_b[b], lsem[b])

        def wait_scatter(b):
            pltpu.make_async_copy(
                out_b[b], acc_sh.at[dst_b[b]], ssem[b]).wait()

            @pl.when(c == 0)
            def _():
                pltpu.make_async_copy(
                    w_b[b], den_sh.at[dst_b[b]], ssem[b]).wait()

        start_chunk(0, 0)
        start_chunk(1, 1)

        def it_body(it, carry):
            for b in range(nb):
                i = it * nb + b

                @pl.when(i < _CH)
                def _():
                    pltpu.make_async_copy(
                        as_h.at[srcu_b[b]], asv_b[b], lsem[b]).wait()
                    pltpu.make_async_copy(
                        ad_h.at[dst_b[b]], adv_b[b], lsem[b]).wait()
                    for g in range(_K // 16):
                        sl = pl.ds(g * 16, 16)
                        z = asv_b[b][sl] + adv_b[b][sl]
                        w_b[b][sl] = jnp.exp(_leaky(z))
                    pltpu.make_async_copy(
                        hcomb_h.at[sg_b[b]], rows_b[b], gsem[b]).wait()
                    @plsc.parallel_loop(0, _K, unroll=8)
                    def _scale(e_i):
                        wspl = plsc.load_gather(
                            w_b[b], [jnp.zeros((16,), jnp.int32) + e_i])
                        for v in range(nv):
                            out_b[b][e_i, pl.ds(v * 16, 16)] = (
                                rows_b[b][e_i, pl.ds(v * 16, 16)] * wspl)
                    b2 = (b + 2) % nb

                    @pl.when((i >= 1) & (i + 2 < _CH))
                    def _():
                        wait_scatter(b2)
                        start_chunk(i + 2, b2)

                    @pl.when(i == 0)
                    def _():
                        start_chunk(2, b2)

                    pltpu.async_copy(
                        out_b[b], acc_sh.at[dst_b[b]], ssem[b], add=True)

                    @pl.when(c == 0)
                    def _():
                        pltpu.async_copy(
                            w_b[b], den_sh.at[dst_b[b]], ssem[b], add=True)
            return carry

        lax.fori_loop(0, (_CH + nb - 1) // nb, it_body, 0)
        for b in range(nb):
            wait_scatter(b)
        plsc.subcore_barrier()

        for cc in range(_NWC):
            @pl.when(s == cc % 16)
            def _():
                pltpu.sync_copy(
                    acc_sh.at[pl.ds(cc * _WCH, _WCH)],
                    acc_h.at[pl.ds(c * N + cc * _WCH, _WCH)])
                pltpu.sync_copy(
                    den_sh.at[pl.ds(cc * _WCH, _WCH)],
                    den_h.at[pl.ds(c * N + cc * _WCH, _WCH)])

    return sc_edge


_sc_edge_enc = _make_sc_edge(D_HID // 2)
_sc_edge_dec = _make_sc_edge(D_IN // 4)


def kernel(x, edges, node_mask, enc_W, enc_a_src, enc_a_dst, enc_bias,
           e2d_W, mask_token, dec_W, dec_a_src, dec_a_dst, dec_bias):
    src = edges[:, 0].astype(jnp.int32)
    dst = edges[:, 1].astype(jnp.int32)
    keepf = node_mask.astype(jnp.float32)[:, None]

    hc1, a1, a2 = _tc_pre(x, keepf, mask_token, enc_W,
                          enc_a_src.reshape(1, D_HID),
                          enc_a_dst.reshape(1, D_HID))
    acc1, den1 = _sc_edge_enc(src, dst, a1.reshape(N), a2.reshape(N),
                              hc1.reshape(2 * N, D_HID // 2))
    hc3, b1, b2 = _tc_mid(acc1.reshape(2, N, D_HID // 2),
                          den1[:N].reshape(N, 1), hc1, a1, a2, keepf,
                          enc_bias.reshape(1, D_HID), e2d_W.T, dec_W,
                          dec_a_src.reshape(1, D_IN),
                          dec_a_dst.reshape(1, D_IN))
    b1f = b1.reshape(N)
    b2f = b2.reshape(N)
    acc2a, den2 = _sc_edge_dec(src, dst, b1f, b2f,
                               hc3[:2].reshape(2 * N, D_IN // 4))
    acc2b, _ = _sc_edge_dec(src, dst, b1f, b2f,
                            hc3[2:].reshape(2 * N, D_IN // 4))
    loss = _tc_loss(acc2a.reshape(2, N, D_IN // 4),
                    acc2b.reshape(2, N, D_IN // 4),
                    den2[:N].reshape(N, 1),
                    hc3, b1, b2, keepf, dec_bias.reshape(1, D_IN), x)
    return loss.reshape(())


# speedup vs baseline: 1.6908x; 1.0203x over previous
import functools

import jax
import jax.numpy as jnp
from jax import lax
from jax.experimental import pallas as pl
from jax.experimental.pallas import tpu as pltpu
from jax.experimental.pallas import tpu_sc as plsc

N = 10000
E = 160000
D_IN = 384
D_HID = 128

_BLK = 1000
_GRID = N // _BLK
_K = 80
_EPT = E // 16
_CH = _EPT // _K
_WCH = 400
_NWC = N // _WCH


def _leaky(z):
    return jnp.where(z >= 0, z, 0.2 * z)


def _tc_pre_body(x_ref, keep_ref, mtok_ref, w_ref, as_ref, ad_ref,
                 hc_ref, a1_ref, a2_ref):
    keep = keep_ref[...]
    xm = keep * x_ref[...] + (1.0 - keep) * mtok_ref[...]
    h = jnp.dot(xm, w_ref[...], preferred_element_type=jnp.float32)
    hc_ref[0] = h[:, : D_HID // 2]
    hc_ref[1] = h[:, D_HID // 2:]
    a1_ref[...] = jnp.sum(h * as_ref[...], axis=1, keepdims=True)
    a2_ref[...] = jnp.sum(h * ad_ref[...], axis=1, keepdims=True)


def _tc_pre(x, keepf, mtok, enc_W, a_s, a_d):
    return pl.pallas_call(
        _tc_pre_body,
        grid=(_GRID,),
        in_specs=[
            pl.BlockSpec((_BLK, D_IN), lambda i: (i, 0)),
            pl.BlockSpec((_BLK, 1), lambda i: (i, 0)),
            pl.BlockSpec((1, D_IN), lambda i: (0, 0)),
            pl.BlockSpec((D_IN, D_HID), lambda i: (0, 0)),
            pl.BlockSpec((1, D_HID), lambda i: (0, 0)),
            pl.BlockSpec((1, D_HID), lambda i: (0, 0)),
        ],
        out_specs=[
            pl.BlockSpec((2, _BLK, D_HID // 2), lambda i: (0, i, 0)),
            pl.BlockSpec((_BLK, 1), lambda i: (i, 0)),
            pl.BlockSpec((_BLK, 1), lambda i: (i, 0)),
        ],
        out_shape=[
            jax.ShapeDtypeStruct((2, N, D_HID // 2), jnp.float32),
            jax.ShapeDtypeStruct((N, 1), jnp.float32),
            jax.ShapeDtypeStruct((N, 1), jnp.float32),
        ],
    )(x, keepf, mtok, enc_W, a_s, a_d)


def _tc_mid_body(acc_ref, den_ref, hc_ref, a1_ref, a2_ref, keep_ref,
                 bias_ref, e2dt_ref, decw_ref, das_ref, dad_ref,
                 h3_ref, b1_ref, b2_ref):
    wself = jnp.exp(_leaky(a1_ref[...] + a2_ref[...]))
    h1 = jnp.concatenate([hc_ref[0], hc_ref[1]], axis=1)
    acc = jnp.concatenate([acc_ref[0], acc_ref[1]], axis=1) + wself * h1
    he = acc / (den_ref[...] + wself + 1e-16) + bias_ref[...]
    hm = keep_ref[...] * he
    h2 = jnp.dot(hm, e2dt_ref[...], preferred_element_type=jnp.float32)
    h3 = jnp.dot(h2, decw_ref[...], preferred_element_type=jnp.float32)
    for q in range(4):
        h3_ref[q] = h3[:, q * (D_IN // 4): (q + 1) * (D_IN // 4)]
    b1_ref[...] = jnp.sum(h3 * das_ref[...], axis=1, keepdims=True)
    b2_ref[...] = jnp.sum(h3 * dad_ref[...], axis=1, keepdims=True)


def _tc_mid(acc1, den1, hc1, a1, a2, keepf, enc_bias, e2d_Wt, dec_W, d_as, d_ad):
    return pl.pallas_call(
        _tc_mid_body,
        grid=(_GRID,),
        in_specs=[
            pl.BlockSpec((2, _BLK, D_HID // 2), lambda i: (0, i, 0)),
            pl.BlockSpec((_BLK, 1), lambda i: (i, 0)),
            pl.BlockSpec((2, _BLK, D_HID // 2), lambda i: (0, i, 0)),
            pl.BlockSpec((_BLK, 1), lambda i: (i, 0)),
            pl.BlockSpec((_BLK, 1), lambda i: (i, 0)),
            pl.BlockSpec((_BLK, 1), lambda i: (i, 0)),
            pl.BlockSpec((1, D_HID), lambda i: (0, 0)),
            pl.BlockSpec((D_HID, D_HID), lambda i: (0, 0)),
            pl.BlockSpec((D_HID, D_IN), lambda i: (0, 0)),
            pl.BlockSpec((1, D_IN), lambda i: (0, 0)),
            pl.BlockSpec((1, D_IN), lambda i: (0, 0)),
        ],
        out_specs=[
            pl.BlockSpec((4, _BLK, D_IN // 4), lambda i: (0, i, 0)),
            pl.BlockSpec((_BLK, 1), lambda i: (i, 0)),
            pl.BlockSpec((_BLK, 1), lambda i: (i, 0)),
        ],
        out_shape=[
            jax.ShapeDtypeStruct((4, N, D_IN // 4), jnp.float32),
            jax.ShapeDtypeStruct((N, 1), jnp.float32),
            jax.ShapeDtypeStruct((N, 1), jnp.float32),
        ],
    )(acc1, den1, hc1, a1, a2, keepf, enc_bias, e2d_Wt, dec_W, d_as, d_ad)


def _tc_loss_body(acca_ref, accb_ref, den_ref, h3_ref, b1_ref, b2_ref,
                  keep_ref, bias_ref, x_ref, loss_ref, s_ref):
    i = pl.program_id(0)

    @pl.when(i == 0)
    def _():
        s_ref[0] = 0.0
        s_ref[1] = 0.0

    wself = jnp.exp(_leaky(b1_ref[...] + b2_ref[...]))
    h3 = jnp.concatenate([h3_ref[q] for q in range(4)], axis=1)
    acc = jnp.concatenate(
        [acca_ref[0], acca_ref[1], accb_ref[0], accb_ref[1]], axis=1
    ) + wself * h3
    xr = acc / (den_ref[...] + wself + 1e-16) + bias_ref[...]
    x = x_ref[...]
    nx = jnp.maximum(jnp.sqrt(jnp.sum(xr * xr, axis=1, keepdims=True)), 1e-12)
    ny = jnp.maximum(jnp.sqrt(jnp.sum(x * x, axis=1, keepdims=True)), 1e-12)
    dot = jnp.sum(xr * x, axis=1, keepdims=True) / (nx * ny)
    z = 1.0 - dot
    mf = 1.0 - keep_ref[...]
    per = z * z * z * mf
    s_ref[0] += jnp.sum(per)
    s_ref[1] += jnp.sum(mf)

    @pl.when(i == _GRID - 1)
    def _():
        loss_ref[...] = jnp.reshape(
            s_ref[0] / jnp.maximum(s_ref[1], 1.0), (1, 1))


def _tc_loss(acc2a, acc2b, den2, hc3, b1, b2, keepf, dec_bias, x):
    return pl.pallas_call(
        _tc_loss_body,
        grid=(_GRID,),
        in_specs=[
            pl.BlockSpec((2, _BLK, D_IN // 4), lambda i: (0, i, 0)),
            pl.BlockSpec((2, _BLK, D_IN // 4), lambda i: (0, i, 0)),
            pl.BlockSpec((_BLK, 1), lambda i: (i, 0)),
            pl.BlockSpec((4, _BLK, D_IN // 4), lambda i: (0, i, 0)),
            pl.BlockSpec((_BLK, 1), lambda i: (i, 0)),
            pl.BlockSpec((_BLK, 1), lambda i: (i, 0)),
            pl.BlockSpec((_BLK, 1), lambda i: (i, 0)),
            pl.BlockSpec((1, D_IN), lambda i: (0, 0)),
            pl.BlockSpec((_BLK, D_IN), lambda i: (i, 0)),
        ],
        out_specs=pl.BlockSpec((1, 1), lambda i: (0, 0)),
        out_shape=jax.ShapeDtypeStruct((1, 1), jnp.float32),
        scratch_shapes=[pltpu.SMEM((2,), jnp.float32)],
    )(acc2a, acc2b, den2, hc3, b1, b2, keepf, dec_bias, x)


def _make_sc_edge(d_half, tables):
    nv = d_half // 16
    nb = 3
    mesh = plsc.VectorSubcoreMesh(core_axis_name="c", subcore_axis_name="s")

    scratch = [
        pltpu.VMEM((_EPT,), jnp.int32),
        pltpu.VMEM((_EPT,), jnp.int32),
        pltpu.VMEM_SHARED((N, d_half), jnp.float32),
        pltpu.VMEM_SHARED((N,), jnp.float32),
    ]
    if tables:
        scratch += [
            pltpu.VMEM((N,), jnp.float32),
            pltpu.VMEM((N,), jnp.float32),
        ]
    for _ in range(nb):
        scratch += [
            pltpu.VMEM((_K,), jnp.int32),
            pltpu.VMEM((_K,), jnp.int32),
            pltpu.VMEM((_K,), jnp.int32),
            pltpu.VMEM((_K,), jnp.float32),
            pltpu.VMEM((_K,), jnp.float32),
            pltpu.VMEM((_K,), jnp.float32),
            pltpu.VMEM((_K, d_half), jnp.float32),
            pltpu.VMEM((_K, d_half), jnp.float32),
            pltpu.SemaphoreType.DMA,
            pltpu.SemaphoreType.DMA,
            pltpu.SemaphoreType.DMA,
        ]

    @functools.partial(
        pl.kernel,
        mesh=mesh,
        compiler_params=pltpu.CompilerParams(
            needs_layout_passes=False, use_tc_tiling_on_sc=False),
        out_type=[
            jax.ShapeDtypeStruct((2 * N, d_half), jnp.float32),
            jax.ShapeDtypeStruct((2 * N,), jnp.float32),
        ],
        scratch_types=scratch,
    )
    def sc_edge(src_h, dst_h, as_h, ad_h, hcomb_h, acc_h, den_h,
                src_t, dst_t, acc_sh, den_sh, *bufs):
        c = lax.axis_index("c")
        s = lax.axis_index("s")
        if tables:
            as_v, ad_v = bufs[0], bufs[1]
            bufs = bufs[2:]
        sg_b = [bufs[11 * b + 0] for b in range(nb)]
        srcu_b = [bufs[11 * b + 1] for b in range(nb)]
        dst_b = [bufs[11 * b + 2] for b in range(nb)]
        asv_b = [bufs[11 * b + 3] for b in range(nb)]
        adv_b = [bufs[11 * b + 4] for b in range(nb)]
        w_b = [bufs[11 * b + 5] for b in range(nb)]
        rows_b = [bufs[11 * b + 6] for b in range(nb)]
        out_b = [bufs[11 * b + 7] for b in range(nb)]
        gsem = [bufs[11 * b + 8] for b in range(nb)]
        lsem = [bufs[11 * b + 9] for b in range(nb)]
        ssem = [bufs[11 * b + 10] for b in range(nb)]

        ebase = s * _EPT
        pltpu.sync_copy(src_h.at[pl.ds(ebase, _EPT)], src_t)
        pltpu.sync_copy(dst_h.at[pl.ds(ebase, _EPT)], dst_t)
        if tables:
            pltpu.sync_copy(as_h, as_v)
            pltpu.sync_copy(ad_h, ad_v)

        zf = jnp.zeros((16,), jnp.float32)
        for r in range(_K):
            for v in range(nv):
                rows_b[0][r, pl.ds(v * 16, 16)] = zf
        for g in range(_K // 16):
            w_b[0][pl.ds(g * 16, 16)] = zf
        for cc in range(_NWC):
            @pl.when(s == cc % 16)
            def _():
                for j in range(_WCH // _K):
                    pltpu.sync_copy(
                        rows_b[0],
                        acc_sh.at[pl.ds(cc * _WCH + j * _K, _K)])
                    pltpu.sync_copy(
                        w_b[0],
                        den_sh.at[pl.ds(cc * _WCH + j * _K, _K)])
        plsc.subcore_barrier()

        off = jnp.full((16,), 0, jnp.int32) + c * N

        def start_chunk(ci, b):
            for g in range(_K // 16):
                sl = pl.ds(ci * _K + g * 16, 16)
                s16 = src_t[sl]
                sg_b[b][pl.ds(g * 16, 16)] = s16 + off
                if not tables:
                    srcu_b[b][pl.ds(g * 16, 16)] = s16
                dst_b[b][pl.ds(g * 16, 16)] = dst_t[sl]
            pltpu.async_copy(hcomb_h.at[sg_b[b]], rows_b[b], gsem[b])
            if not tables:
                pltpu.async_copy(as_h.at[srcu_b[b]], asv_b[b], lsem[b])
                pltpu.async_copy(ad_h.at[dst_b[b]], adv_b[b], lsem[b])

        def wait_scatter(b):
            pltpu.make_async_copy(
                out_b[b], acc_sh.at[dst_b[b]], ssem[b]).wait()

            @pl.when(c == 0)
            def _():
                pltpu.make_async_copy(
                    w_b[b], den_sh.at[dst_b[b]], ssem[b]).wait()

        start_chunk(0, 0)
        start_chunk(1, 1)

        def it_body(it, carry):
            for b in range(nb):
                i = it * nb + b

                @pl.when(i < _CH)
                def _():
                    if tables:
                        for g in range(_K // 16):
                            sl = pl.ds(i * _K + g * 16, 16)
                            z = (plsc.load_gather(as_v, [src_t[sl]])
                                 + plsc.load_gather(ad_v, [dst_t[sl]]))
                            w_b[b][pl.ds(g * 16, 16)] = jnp.exp(_leaky(z))
                    else:
                        pltpu.make_async_copy(
                            as_h.at[srcu_b[b]], asv_b[b], lsem[b]).wait()
                        pltpu.make_async_copy(
                            ad_h.at[dst_b[b]], adv_b[b], lsem[b]).wait()
                        for g in range(_K // 16):
                            sl = pl.ds(g * 16, 16)
                            z = asv_b[b][sl] + adv_b[b][sl]
                            w_b[b][sl] = jnp.exp(_leaky(z))
                    pltpu.make_async_copy(
                        hcomb_h.at[sg_b[b]], rows_b[b], gsem[b]).wait()
                    @plsc.parallel_loop(0, _K, unroll=16)
                    def _scale(e_i):
                        wspl = plsc.load_gather(
                            w_b[b], [jnp.zeros((16,), jnp.int32) + e_i])
                        for v in range(nv):
                            out_b[b][e_i, pl.ds(v * 16, 16)] = (
                                rows_b[b][e_i, pl.ds(v * 16, 16)] * wspl)
                    b2 = (b + 2) % nb

                    @pl.when((i >= 1) & (i + 2 < _CH))
                    def _():
                        wait_scatter(b2)
                        start_chunk(i + 2, b2)

                    @pl.when(i == 0)
                    def _():
                        start_chunk(2, b2)

                    pltpu.async_copy(
                        out_b[b], acc_sh.at[dst_b[b]], ssem[b], add=True)

                    @pl.when(c == 0)
                    def _():
                        pltpu.async_copy(
                            w_b[b], den_sh.at[dst_b[b]], ssem[b], add=True)
            return carry

        lax.fori_loop(0, (_CH + nb - 1) // nb, it_body, 0)
        for b in range(nb):
            wait_scatter(b)
        plsc.subcore_barrier()

        for cc in range(_NWC):
            @pl.when(s == cc % 16)
            def _():
                pltpu.sync_copy(
                    acc_sh.at[pl.ds(cc * _WCH, _WCH)],
                    acc_h.at[pl.ds(c * N + cc * _WCH, _WCH)])
                pltpu.sync_copy(
                    den_sh.at[pl.ds(cc * _WCH, _WCH)],
                    den_h.at[pl.ds(c * N + cc * _WCH, _WCH)])

    return sc_edge


_sc_edge_enc = _make_sc_edge(D_HID // 2, tables=True)
_sc_edge_dec = _make_sc_edge(D_IN // 4, tables=False)


def kernel(x, edges, node_mask, enc_W, enc_a_src, enc_a_dst, enc_bias,
           e2d_W, mask_token, dec_W, dec_a_src, dec_a_dst, dec_bias):
    src = edges[:, 0].astype(jnp.int32)
    dst = edges[:, 1].astype(jnp.int32)
    keepf = node_mask.astype(jnp.float32)[:, None]

    hc1, a1, a2 = _tc_pre(x, keepf, mask_token, enc_W,
                          enc_a_src.reshape(1, D_HID),
                          enc_a_dst.reshape(1, D_HID))
    acc1, den1 = _sc_edge_enc(src, dst, a1.reshape(N), a2.reshape(N),
                              hc1.reshape(2 * N, D_HID // 2))
    hc3, b1, b2 = _tc_mid(acc1.reshape(2, N, D_HID // 2),
                          den1[:N].reshape(N, 1), hc1, a1, a2, keepf,
                          enc_bias.reshape(1, D_HID), e2d_W.T, dec_W,
                          dec_a_src.reshape(1, D_IN),
                          dec_a_dst.reshape(1, D_IN))
    b1f = b1.reshape(N)
    b2f = b2.reshape(N)
    acc2a, den2 = _sc_edge_dec(src, dst, b1f, b2f,
                               hc3[:2].reshape(2 * N, D_IN // 4))
    acc2b, _ = _sc_edge_dec(src, dst, b1f, b2f,
                            hc3[2:].reshape(2 * N, D_IN // 4))
    loss = _tc_loss(acc2a.reshape(2, N, D_IN // 4),
                    acc2b.reshape(2, N, D_IN // 4),
                    den2[:N].reshape(N, 1),
                    hc3, b1, b2, keepf, dec_bias.reshape(1, D_IN), x)
    return loss.reshape(())
